# Initial kernel scaffold; baseline (speedup 1.0000x reference)
import jax
import jax.numpy as jnp
from jax.experimental import pallas as pl


def kernel(x):
    raise NotImplementedError("write your pallas kernel here")


# speedup vs baseline: 1.5347x; 1.5347x over previous
import jax
import jax.numpy as jnp
from jax.experimental import pallas as pl
from jax.experimental.pallas import tpu as pltpu

_EPS = 0.1


def _zono_body(x_ref, o_ref):
    pv = x_ref[0]
    low = pv < _EPS
    high = pv > 1.0 - _EPS
    new_pv = jnp.where(low, (pv + _EPS) * 0.5,
             jnp.where(high, (pv + 1.0 - _EPS) * 0.5, pv))
    new_e = jnp.where(low, (_EPS + pv) * 0.5,
            jnp.where(high, (1.0 - pv + _EPS) * 0.5, jnp.full_like(pv, _EPS)))
    c, hw = o_ref.shape[1], o_ref.shape[2]
    row = jax.lax.broadcasted_iota(jnp.int32, (c, hw), 0)
    col = jax.lax.broadcasted_iota(jnp.int32, (c, hw), 1)
    out = jnp.where(row == 0, new_pv, jnp.where(row == col + 1, new_e, 0.0))
    o_ref[0] = out


def kernel(x):
    B, _, H, W = x.shape
    HW = H * W
    xf = x.reshape(B, 1, HW)
    out = pl.pallas_call(
        _zono_body,
        grid=(B,),
        in_specs=[pl.BlockSpec((1, 1, HW), lambda b: (b, 0, 0))],
        out_specs=pl.BlockSpec((1, 1 + HW, HW), lambda b: (b, 0, 0)),
        out_shape=jax.ShapeDtypeStruct((B, 1 + HW, HW), x.dtype),
        compiler_params=pltpu.CompilerParams(
            dimension_semantics=("parallel",),
        ),
    )(xf)
    return out.reshape(B, 1 + HW, H, W)
